# baseline (speedup 1.0000x reference)
import functools

import jax
import jax.numpy as jnp
from jax import lax
from jax.experimental import pallas as pl
from jax.experimental.pallas import tpu as pltpu
from jax.experimental.pallas import tpu_sc as plsc

_NC = 2
_NS = 16
_NW = _NC * _NS


def _make_sc_kernel(s, b, d, rows_per_w, ch):
    nchunk = rows_per_w // ch
    mesh = plsc.VectorSubcoreMesh(
        core_axis_name="c", subcore_axis_name="s",
        num_cores=_NC, num_subcores=_NS)

    @functools.partial(
        pl.kernel,
        out_type=jax.ShapeDtypeStruct((s, b * d), jnp.float32),
        mesh=mesh,
        scratch_types=[
            pltpu.VMEM((ch, d), jnp.float32),
            pltpu.VMEM((ch, d), jnp.float32),
            pltpu.SemaphoreType.DMA,
            pltpu.SemaphoreType.DMA,
            pltpu.SemaphoreType.DMA,
        ],
    )
    def sc_copy(table_hbm, out_hbm, buf0, buf1, sem0, sem1, sem_out):
        wid = lax.axis_index("s") * _NC + lax.axis_index("c")
        base = wid * rows_per_w
        bufs = (buf0, buf1)
        sems = (sem0, sem1)
        in_copies = [None, None]
        in_copies[0] = pltpu.async_copy(
            table_hbm.at[pl.ds(base, ch)], buf0, sem0)
        for c in range(nchunk):
            cur = c % 2
            if c + 1 < nchunk:
                nxt = (c + 1) % 2
                in_copies[nxt] = pltpu.async_copy(
                    table_hbm.at[pl.ds(base + (c + 1) * ch, ch)],
                    bufs[nxt], sems[nxt])
            in_copies[cur].wait()
            row0 = base + c * ch
            out_copies = []
            for j in range(b):
                out_copies.append(pltpu.async_copy(
                    bufs[cur],
                    out_hbm.at[pl.ds(row0, ch), pl.ds(j * d, d)],
                    sem_out))
            for cp in out_copies:
                cp.wait()

    return sc_copy


def kernel(x, table):
    s = x.shape[0]
    b = x.shape[1]
    d = table.shape[1]
    rows_per_w = s // _NW
    ch = min(32, rows_per_w)
    out2d = _make_sc_kernel(s, b, d, rows_per_w, ch)(table)
    return out2d.reshape(s, b, d)
